# Initial kernel scaffold; baseline (speedup 1.0000x reference)
import jax
import jax.numpy as jnp
from jax.experimental import pallas as pl


def kernel(input_ids, attention_mask, position_ids, control_classes, labels, W):
    raise NotImplementedError("write your pallas kernel here")


# speedup vs baseline: 1.3597x; 1.3597x over previous
import functools
import math

import jax
import jax.numpy as jnp
import numpy as np
from jax import lax
from jax.experimental import pallas as pl
from jax.experimental.pallas import tpu as pltpu
from jax.experimental.pallas import tpu_sc as plsc

HIDDEN = 2048
HEAD_DIM = 128
ROPE_THETA = 10000.0
_NORM = np.float32(float(HIDDEN) ** 0.5)
_F32_MIN = float(np.finfo(np.float32).min)

_NC = 2
_NS = 16
_NW = _NC * _NS


def _make_sc_gather(n_tokens: int):
    per_w = n_tokens // _NW
    C = 8
    nchunks = per_w // C
    mesh = plsc.VectorSubcoreMesh(core_axis_name="c", subcore_axis_name="s")

    @functools.partial(
        pl.kernel,
        mesh=mesh,
        out_type=jax.ShapeDtypeStruct((n_tokens, HIDDEN), jnp.float32),
        scratch_types=[
            pltpu.VMEM((per_w,), jnp.int32),
            pltpu.VMEM((C, HIDDEN), jnp.float32),
            pltpu.SemaphoreType.DMA,
        ],
    )
    def sc_gather(ids_hbm, w_hbm, out_hbm, idx_v, buf, sem):
        wid = lax.axis_index("s") * _NC + lax.axis_index("c")
        base = wid * per_w
        pltpu.sync_copy(ids_hbm.at[pl.ds(base, per_w)], idx_v)

        def chunk_body(c):
            idx_sl = idx_v.at[pl.ds(c * C, C)]
            pltpu.async_copy(w_hbm.at[idx_sl], buf, sem).wait()

            def scale_row(r, carry):
                for k in range(HIDDEN // 16):
                    sl = (r, pl.ds(k * 16, 16))
                    buf[sl] = buf[sl] * _NORM
                return carry

            lax.fori_loop(0, C, scale_row, 0)
            pltpu.sync_copy(buf, out_hbm.at[pl.ds(base + c * C, C)])

        pl.loop(0, nchunks)(chunk_body)

    return sc_gather


def _mask_body(am_ref, out_ref):
    i = pl.program_id(1)
    bs, s = out_ref.shape[1], out_ref.shape[2]
    rows = lax.broadcasted_iota(jnp.int32, (bs, s), 0) + i * bs
    cols = lax.broadcasted_iota(jnp.int32, (bs, s), 1)
    masked = (cols > rows) | (am_ref[0] == 0)
    out_ref[0] = jnp.where(masked, jnp.float32(_F32_MIN), jnp.float32(0.0))


def _make_mask(batch: int, s: int):
    bs = 512
    return pl.pallas_call(
        _mask_body,
        grid=(batch, s // bs),
        in_specs=[pl.BlockSpec((1, 1, s), lambda b, i: (b, 0, 0))],
        out_specs=pl.BlockSpec((1, bs, s), lambda b, i: (b, i, 0)),
        out_shape=jax.ShapeDtypeStruct((batch, s, s), jnp.float32),
    )


def _rope_body(pos_ref, cos_ref, sin_ref):
    p = pos_ref[:, :].astype(jnp.float32)
    k = lax.broadcasted_iota(jnp.int32, (1, HEAD_DIM // 2), 1).astype(jnp.float32)
    inv = jnp.exp(k * jnp.float32(-2.0 * math.log(ROPE_THETA) / HEAD_DIM))
    freqs = p * inv
    emb = jnp.concatenate([freqs, freqs], axis=-1)
    cos_ref[:, :] = jnp.cos(emb)
    sin_ref[:, :] = jnp.sin(emb)


def _make_rope(s: int):
    return pl.pallas_call(
        _rope_body,
        out_shape=[
            jax.ShapeDtypeStruct((s, HEAD_DIM), jnp.float32),
            jax.ShapeDtypeStruct((s, HEAD_DIM), jnp.float32),
        ],
    )


def kernel(input_ids, attention_mask, position_ids, control_classes, labels, W):
    b, s = input_ids.shape
    ids = input_ids.reshape(-1).astype(jnp.int32)

    emb = _make_sc_gather(b * s)(ids, W)
    hidden = emb.reshape(b, s, HIDDEN)

    mask = _make_mask(b, s)(attention_mask.reshape(b, 1, s)).reshape(b, 1, s, s)

    cos2, sin2 = _make_rope(s)(position_ids.reshape(s, 1))
    cos = cos2.reshape(1, s, HEAD_DIM)
    sin = sin2.reshape(1, s, HEAD_DIM)

    cache_position = jnp.arange(0, s)
    return (hidden, mask, cos, sin, cache_position, control_classes, labels)


# speedup vs baseline: 1.4944x; 1.0990x over previous
import functools
import math

import jax
import jax.numpy as jnp
import numpy as np
from jax import lax
from jax.experimental import pallas as pl
from jax.experimental.pallas import tpu as pltpu
from jax.experimental.pallas import tpu_sc as plsc

HIDDEN = 2048
HEAD_DIM = 128
ROPE_THETA = 10000.0
_NORM = np.float32(float(HIDDEN) ** 0.5)
_F32_MIN = float(np.finfo(np.float32).min)

_NC = 2
_NS = 16
_NW = _NC * _NS


def _make_sc_gather(n_tokens: int):
    per_w = n_tokens // _NW
    C = 8
    NBUF = 2
    nchunks = per_w // C
    mesh = plsc.VectorSubcoreMesh(core_axis_name="c", subcore_axis_name="s")

    @functools.partial(
        pl.kernel,
        mesh=mesh,
        out_type=jax.ShapeDtypeStruct((n_tokens, HIDDEN), jnp.float32),
        scratch_types=[
            pltpu.VMEM((per_w,), jnp.int32),
            pltpu.VMEM((C, HIDDEN), jnp.float32),
            pltpu.VMEM((C, HIDDEN), jnp.float32),
            pltpu.VMEM((C, HIDDEN), jnp.float32),
            pltpu.VMEM((C, HIDDEN), jnp.float32),
            pltpu.SemaphoreType.DMA,
            pltpu.SemaphoreType.DMA,
            pltpu.SemaphoreType.DMA,
            pltpu.SemaphoreType.DMA,
        ],
    )
    def sc_gather(ids_hbm, w_hbm, out_hbm, idx_v,
                  in0, in1, ot0, ot1, si0, si1, so0, so1):
        inb, outb, sin, son = [in0, in1], [ot0, ot1], [si0, si1], [so0, so1]
        wid = lax.axis_index("s") * _NC + lax.axis_index("c")
        base = wid * per_w
        pltpu.sync_copy(ids_hbm.at[pl.ds(base, per_w)], idx_v)

        def gather_start(c, b):
            pltpu.make_async_copy(
                w_hbm.at[idx_v.at[pl.ds(c * C, C)]], inb[b], sin[b]).start()

        def out_copy(c, b):
            return pltpu.make_async_copy(
                outb[b], out_hbm.at[pl.ds(base + c * C, C)], son[b])

        for b in range(NBUF):
            gather_start(b, b)

        def body(c0):
            for b in range(NBUF):
                c = c0 + b
                pltpu.make_async_copy(
                    w_hbm.at[idx_v.at[pl.ds(c * C, C)]], inb[b], sin[b]).wait()

                @pl.when(c >= NBUF)
                def _():
                    out_copy(c, b).wait()

                def scale_row(r, carry):
                    for k in range(HIDDEN // 16):
                        outb[b][r, pl.ds(k * 16, 16)] = (
                            inb[b][r, pl.ds(k * 16, 16)] * _NORM)
                    return carry

                lax.fori_loop(0, C, scale_row, 0)

                @pl.when(c + NBUF < nchunks)
                def _():
                    gather_start(c + NBUF, b)

                out_copy(c, b).start()

        pl.loop(0, nchunks, step=NBUF)(body)
        for b in range(NBUF):
            out_copy(nchunks - NBUF + b, b).wait()

    return sc_gather


def _mask_body(am_ref, out_ref):
    i = pl.program_id(1)
    bs, s = out_ref.shape[1], out_ref.shape[2]
    rows = lax.broadcasted_iota(jnp.int32, (bs, s), 0) + i * bs
    cols = lax.broadcasted_iota(jnp.int32, (bs, s), 1)
    masked = (cols > rows) | (am_ref[0] == 0)
    out_ref[0] = jnp.where(masked, jnp.float32(_F32_MIN), jnp.float32(0.0))


def _make_mask(batch: int, s: int):
    bs = 512
    return pl.pallas_call(
        _mask_body,
        grid=(batch, s // bs),
        in_specs=[pl.BlockSpec((1, 1, s), lambda b, i: (b, 0, 0))],
        out_specs=pl.BlockSpec((1, bs, s), lambda b, i: (b, i, 0)),
        out_shape=jax.ShapeDtypeStruct((batch, s, s), jnp.float32),
    )


def _rope_body(pos_ref, cos_ref, sin_ref):
    p = pos_ref[:, :].astype(jnp.float32)
    k = lax.broadcasted_iota(jnp.int32, (1, HEAD_DIM // 2), 1).astype(jnp.float32)
    inv = jnp.exp(k * jnp.float32(-2.0 * math.log(ROPE_THETA) / HEAD_DIM))
    freqs = p * inv
    emb = jnp.concatenate([freqs, freqs], axis=-1)
    cos_ref[:, :] = jnp.cos(emb)
    sin_ref[:, :] = jnp.sin(emb)


def _make_rope(s: int):
    return pl.pallas_call(
        _rope_body,
        out_shape=[
            jax.ShapeDtypeStruct((s, HEAD_DIM), jnp.float32),
            jax.ShapeDtypeStruct((s, HEAD_DIM), jnp.float32),
        ],
    )


def kernel(input_ids, attention_mask, position_ids, control_classes, labels, W):
    b, s = input_ids.shape
    ids = input_ids.reshape(-1).astype(jnp.int32)

    emb = _make_sc_gather(b * s)(ids, W)
    hidden = emb.reshape(b, s, HIDDEN)

    mask = _make_mask(b, s)(attention_mask.reshape(b, 1, s)).reshape(b, 1, s, s)

    cos2, sin2 = _make_rope(s)(position_ids.reshape(s, 1))
    cos = cos2.reshape(1, s, HEAD_DIM)
    sin = sin2.reshape(1, s, HEAD_DIM)

    cache_position = jnp.arange(0, s)
    return (hidden, mask, cos, sin, cache_position, control_classes, labels)
